# baseline (speedup 1.0000x reference)
import functools

import jax
import jax.numpy as jnp
from jax import lax
from jax.experimental import pallas as pl
from jax.experimental.pallas import tpu as pltpu
from jax.experimental.pallas import tpu_sc as plsc

SEQ_SIZE = 200
EMBED_DIM = 64
BATCH = 4096


def _make_sc_gather(V, D, B, NW, b_per_w):
    mesh = plsc.VectorSubcoreMesh(core_axis_name="c", subcore_axis_name="s")

    NB = 12

    @functools.partial(
        pl.kernel,
        mesh=mesh,
        out_type=jax.ShapeDtypeStruct((D, B), jnp.float32),
        scratch_types=[
            pltpu.VMEM((b_per_w,), jnp.int32),
            pltpu.VMEM((NB, D, 128), jnp.float32),
            pltpu.VMEM((D, b_per_w), jnp.float32),
            pltpu.SemaphoreType.DMA,
        ],
        compiler_params=pltpu.CompilerParams(
            needs_layout_passes=False, disable_bounds_checks=True),
    )
    def gather_kernel(table_hbm, idx_hbm, out_hbm, idx_v, slabs,
                      rows_t, sem):
        wid = lax.axis_index("s") * 2 + lax.axis_index("c")
        base = wid * b_per_w
        pltpu.sync_copy(idx_hbm.at[pl.ds(base, b_per_w)], idx_v)
        iota16 = lax.iota(jnp.int32, 16)

        def scalar_idx(b):
            chunk = idx_v[pl.ds((b // 16) * 16, 16)]
            sel = jnp.where(iota16 == (b % 16), chunk, 0)
            return jnp.sum(sel)

        def start(b):
            c0 = (scalar_idx(b) // 128) * 128
            return pltpu.async_copy(
                table_hbm.at[:, pl.ds(pl.multiple_of(c0, 128), 128)],
                slabs.at[b % NB], sem)

        def extract(b):
            lane = scalar_idx(b) % 128
            col = jnp.full((16,), lane, jnp.int32)
            dstcol = jnp.full((16,), b, jnp.int32)
            for g in range(D // 16):
                v = plsc.load_gather(slabs.at[b % NB], [iota16 + g * 16, col])
                plsc.store_scatter(rows_t, [iota16 + g * 16, dstcol], v)

        copies = [start(b) for b in range(NB)]
        for b in range(b_per_w):
            copies[b].wait()
            extract(b)
            if b + NB < b_per_w:
                copies.append(start(b + NB))
        pltpu.sync_copy(rows_t, out_hbm.at[:, pl.ds(base, b_per_w)])

    return gather_kernel


def _bcast_add_body(g_ref, posb_ref, out_ref):
    g = g_ref[...]
    pb = posb_ref[...]
    out_ref[...] = pb + g[None, :, :]


def kernel(inputs, token_table, pos_table):
    V, D = token_table.shape
    B = inputs.shape[0]
    info = plsc.get_sparse_core_info()
    NW = info.num_cores * info.num_subcores
    b_per_w = B // NW

    idx = inputs.reshape(B).astype(jnp.int32)
    gT = _make_sc_gather(V, D, B, NW, b_per_w)(token_table.T, idx)

    BBL = 256
    posB = jnp.broadcast_to(pos_table[:, :, None], (SEQ_SIZE, D, BBL))
    P = pl.pallas_call(
        _bcast_add_body,
        grid=(B // BBL,),
        in_specs=[
            pl.BlockSpec((D, BBL), lambda i: (0, i)),
            pl.BlockSpec((SEQ_SIZE, D, BBL), lambda i: (0, 0, 0)),
        ],
        out_specs=pl.BlockSpec((SEQ_SIZE, D, BBL), lambda i: (0, 0, i)),
        out_shape=jax.ShapeDtypeStruct((SEQ_SIZE, D, B), jnp.float32),
    )(gT, posB)
    return jnp.transpose(P, (2, 0, 1))


# speedup vs baseline: 1.0069x; 1.0069x over previous
import functools

import jax
import jax.numpy as jnp
from jax import lax
from jax.experimental import pallas as pl
from jax.experimental.pallas import tpu as pltpu
from jax.experimental.pallas import tpu_sc as plsc

SEQ_SIZE = 200
EMBED_DIM = 64
BATCH = 4096


def _make_sc_gather(V, D, B, NW, b_per_w):
    mesh = plsc.VectorSubcoreMesh(core_axis_name="c", subcore_axis_name="s")

    NB = 8

    @functools.partial(
        pl.kernel,
        mesh=mesh,
        out_type=jax.ShapeDtypeStruct((D, B), jnp.float32),
        scratch_types=[
            pltpu.VMEM((b_per_w,), jnp.int32),
            pltpu.VMEM((NB, D, 128), jnp.float32),
            pltpu.VMEM((D, b_per_w), jnp.float32),
            pltpu.SemaphoreType.DMA,
        ],
        compiler_params=pltpu.CompilerParams(
            needs_layout_passes=False, disable_bounds_checks=True),
    )
    def gather_kernel(table_hbm, idx_hbm, out_hbm, idx_v, slabs,
                      rows_t, sem):
        wid = lax.axis_index("s") * 2 + lax.axis_index("c")
        base = wid * b_per_w
        pltpu.sync_copy(idx_hbm.at[pl.ds(base, b_per_w)], idx_v)
        iota16 = lax.iota(jnp.int32, 16)

        def scalar_idx(b):
            chunk = idx_v[pl.ds((b // 16) * 16, 16)]
            sel = jnp.where(iota16 == (b % 16), chunk, 0)
            return jnp.sum(sel)

        def start(b):
            c0 = (scalar_idx(b) // 128) * 128
            return pltpu.async_copy(
                table_hbm.at[:, pl.ds(pl.multiple_of(c0, 128), 128)],
                slabs.at[b % NB], sem)

        def extract(b):
            lane = scalar_idx(b) % 128
            col = jnp.full((16,), lane, jnp.int32)
            dstcol = jnp.full((16,), b, jnp.int32)
            for g in range(D // 16):
                v = plsc.load_gather(slabs.at[b % NB], [iota16 + g * 16, col])
                plsc.store_scatter(rows_t, [iota16 + g * 16, dstcol], v)

        copies = [start(b) for b in range(NB)]
        for b in range(b_per_w):
            copies[b].wait()
            extract(b)
            if b + NB < b_per_w:
                copies.append(start(b + NB))
        pltpu.sync_copy(rows_t, out_hbm.at[:, pl.ds(base, b_per_w)])

    return gather_kernel


def _bcast_add_body(g_ref, posb_ref, out_ref):
    g = g_ref[...]
    pb = posb_ref[...]
    out_ref[...] = pb + g[None, :, :]


def kernel(inputs, token_table, pos_table):
    V, D = token_table.shape
    B = inputs.shape[0]
    info = plsc.get_sparse_core_info()
    NW = info.num_cores * info.num_subcores
    b_per_w = B // NW

    idx = inputs.reshape(B).astype(jnp.int32)
    gT = _make_sc_gather(V, D, B, NW, b_per_w)(token_table.T, idx)

    BBL = 256
    posB = jnp.broadcast_to(pos_table[:, :, None], (SEQ_SIZE, D, BBL))
    P = pl.pallas_call(
        _bcast_add_body,
        grid=(B // BBL,),
        in_specs=[
            pl.BlockSpec((D, BBL), lambda i: (0, i)),
            pl.BlockSpec((SEQ_SIZE, D, BBL), lambda i: (0, 0, 0)),
        ],
        out_specs=pl.BlockSpec((SEQ_SIZE, D, BBL), lambda i: (0, 0, i)),
        out_shape=jax.ShapeDtypeStruct((SEQ_SIZE, D, B), jnp.float32),
    )(gT, posB)
    return jnp.transpose(P, (2, 0, 1))
